# baseline (speedup 1.0000x reference)
import functools

import jax
import jax.numpy as jnp
from jax import lax
from jax.experimental import pallas as pl
from jax.experimental.pallas import tpu as pltpu
from jax.experimental.pallas import tpu_sc as plsc

N = 10000
E = 320000
D = 128
NC = 2
NS = 16
NW = NC * NS
EPW = E // NW
CH = 64
IB = 26
NIB = 6
NCH = NIB * IB
TE = EPW - NCH * CH
RPT = 624
TAIL = N - NS * RPT


def _matmul_body(x_ref, w_ref, o_ref):
    o_ref[...] = lax.dot_general(
        x_ref[...], w_ref[...], (((1,), (1,)), ((), ())),
        preferred_element_type=jnp.float32)


def _project(x, W):
    return pl.pallas_call(
        _matmul_body,
        grid=(10,),
        in_specs=[
            pl.BlockSpec((N // 10, D), lambda i: (i, 0)),
            pl.BlockSpec((D, D), lambda i: (0, 0)),
        ],
        out_specs=pl.BlockSpec((N // 10, D), lambda i: (i, 0)),
        out_shape=jax.ShapeDtypeStruct((N, D), jnp.float32),
    )(x, W)


_mesh = plsc.VectorSubcoreMesh(
    core_axis_name="c", subcore_axis_name="s", num_cores=NC, num_subcores=NS)


@functools.partial(
    pl.kernel,
    out_type=jax.ShapeDtypeStruct((NC, N, D), jnp.float32),
    mesh=_mesh,
    scratch_types=[
        pltpu.VMEM((IB, CH), jnp.int32),
        pltpu.VMEM((IB, CH), jnp.int32),
        pltpu.VMEM((1, TE), jnp.int32),
        pltpu.VMEM((1, TE), jnp.int32),
        pltpu.VMEM((2, CH, D), jnp.float32),
        pltpu.VMEM((2, CH, D), jnp.float32),
        pltpu.VMEM_SHARED((N, D), jnp.float32),
        pltpu.SemaphoreType.DMA((2,)),
        pltpu.SemaphoreType.DMA((2,)),
        pltpu.SemaphoreType.DMA((2,)),
    ],
)
def _message_pass(xw_hbm, srcm_hbm, dstm_hbm, srct_hbm, dstt_hbm, ea_hbm,
                  out_hbm, src_v, dst_v, srct_v, dstt_v, xj_v, em_v, acc,
                  gsem, esem, ssem):
    c = lax.axis_index("c")
    s = lax.axis_index("s")
    wid = s * NC + c
    ebase = wid * EPW

    zero16 = jnp.zeros((16,), jnp.float32)

    def zero_body(r, zcarry):
        for k in range(D // 16):
            em_v[0, r, pl.ds(k * 16, 16)] = zero16
        return zcarry

    lax.fori_loop(0, CH, zero_body, 0)
    for i in range(RPT // CH):
        pltpu.sync_copy(em_v.at[0], acc.at[pl.ds(s * RPT + i * CH, CH)])
    rem = RPT - (RPT // CH) * CH
    pltpu.sync_copy(em_v.at[0, pl.ds(0, rem)],
                    acc.at[pl.ds(s * RPT + RPT - rem, rem)])

    @pl.when(s == 0)
    def _():
        pltpu.sync_copy(em_v.at[0, pl.ds(0, TAIL)],
                        acc.at[pl.ds(NS * RPT, TAIL)])

    plsc.subcore_barrier()

    def _wait_gather(p):
        pltpu.make_async_copy(
            xw_hbm.at[pl.ds(0, CH)], xj_v.at[p], gsem.at[p]).wait()

    def _wait_ea(p):
        pltpu.make_async_copy(
            ea_hbm.at[pl.ds(0, CH)], em_v.at[p], esem.at[p]).wait()

    def _wait_scatter(p):
        pltpu.make_async_copy(
            em_v.at[p], acc.at[pl.ds(0, CH)], ssem.at[p]).wait()

    def _issue_ea(j):
        pltpu.async_copy(ea_hbm.at[pl.ds(ebase + j * CH, CH)],
                         em_v.at[lax.rem(j, 2)], esem.at[lax.rem(j, 2)])

    def blk_body(bi, bcarry):
        j0 = bi * IB

        @pl.when(bi > 0)
        def _():
            _wait_scatter(lax.rem(j0 - 1, 2))

        pltpu.sync_copy(srcm_hbm.at[wid, bi], src_v)
        pltpu.sync_copy(dstm_hbm.at[wid, bi], dst_v)

        p0 = lax.rem(j0, 2)
        pltpu.async_copy(xw_hbm.at[src_v.at[0]], xj_v.at[p0], gsem.at[p0])

        @pl.when(bi == 0)
        def _():
            _issue_ea(0)

        def chunk_body(jj, carry):
            j = j0 + jj
            p = lax.rem(j, 2)
            q = 1 - p

            @pl.when(jj + 1 < IB)
            def _():
                pltpu.async_copy(xw_hbm.at[src_v.at[jj + 1]],
                                 xj_v.at[q], gsem.at[q])

            _wait_gather(p)
            _wait_ea(p)

            def row_body(r, rcarry):
                for k in range(D // 16):
                    sl = pl.ds(k * 16, 16)
                    em_v[p, r, sl] = jnp.maximum(
                        xj_v[p, r, sl] + em_v[p, r, sl], 0.0)
                return rcarry

            lax.fori_loop(0, CH, row_body, 0)

            @pl.when(jj >= 1)
            def _():
                _wait_scatter(q)

            @pl.when(j + 1 < NCH)
            def _():
                _issue_ea(j + 1)

            pltpu.async_copy(em_v.at[p], acc.at[dst_v.at[jj]],
                             ssem.at[p], add=True)
            return carry

        lax.fori_loop(0, IB, chunk_body, 0)
        return bcarry

    lax.fori_loop(0, NIB, blk_body, 0)
    _wait_scatter((NCH - 1) % 2)

    pltpu.sync_copy(srct_hbm.at[wid], srct_v)
    pltpu.sync_copy(dstt_hbm.at[wid], dstt_v)
    tg = pltpu.async_copy(xw_hbm.at[srct_v.at[0]],
                          xj_v.at[0, pl.ds(0, TE)], gsem.at[0])
    pltpu.sync_copy(ea_hbm.at[pl.ds(ebase + NCH * CH, TE)],
                    em_v.at[0, pl.ds(0, TE)])
    tg.wait()

    def tail_body(r, rcarry):
        for k in range(D // 16):
            sl = pl.ds(k * 16, 16)
            em_v[0, r, sl] = jnp.maximum(xj_v[0, r, sl] + em_v[0, r, sl],
                                         0.0)
        return rcarry

    lax.fori_loop(0, TE, tail_body, 0)
    pltpu.sync_copy(em_v.at[0, pl.ds(0, TE)], acc.at[dstt_v.at[0]],
                    add=True)
    plsc.subcore_barrier()

    pltpu.sync_copy(acc.at[pl.ds(s * RPT, RPT)],
                    out_hbm.at[c, pl.ds(s * RPT, RPT)])

    @pl.when(s == 0)
    def _():
        pltpu.sync_copy(acc.at[pl.ds(NS * RPT, TAIL)],
                        out_hbm.at[c, pl.ds(NS * RPT, TAIL)])


def _combine_body(p_ref, b_ref, o_ref):
    o_ref[...] = p_ref[0] + p_ref[1] + b_ref[...]


def _combine(partials, b2d):
    return pl.pallas_call(
        _combine_body,
        grid=(10,),
        in_specs=[
            pl.BlockSpec((NC, N // 10, D), lambda i: (0, i, 0)),
            pl.BlockSpec((1, D), lambda i: (0, 0)),
        ],
        out_specs=pl.BlockSpec((N // 10, D), lambda i: (i, 0)),
        out_shape=jax.ShapeDtypeStruct((N, D), jnp.float32),
    )(partials, b2d)


def kernel(x, edge_index, edge_attr, W, b):
    src = edge_index[0].reshape(NW, EPW)
    dst = edge_index[1].reshape(NW, EPW)
    srcm = src[:, :NCH * CH].reshape(NW, NIB, IB, CH)
    dstm = dst[:, :NCH * CH].reshape(NW, NIB, IB, CH)
    srct = src[:, NCH * CH:].reshape(NW, 1, TE)
    dstt = dst[:, NCH * CH:].reshape(NW, 1, TE)
    xw = _project(x, W)
    partials = _message_pass(xw, srcm, dstm, srct, dstt, edge_attr)
    return _combine(partials, b.reshape(1, D))


# speedup vs baseline: 2.1379x; 2.1379x over previous
import functools

import jax
import jax.numpy as jnp
from jax import lax
from jax.experimental import pallas as pl
from jax.experimental.pallas import tpu as pltpu
from jax.experimental.pallas import tpu_sc as plsc

N = 10000
E = 320000
D = 128
NC = 2
NS = 16
NW = NC * NS
EPW = E // NW
CH = 64
IB = 26
NP = IB // 2
NIB = 6
NCH = NIB * IB
TE = EPW - NCH * CH
RPT = 624
TAIL = N - NS * RPT


def _matmul_body(x_ref, w_ref, o_ref):
    o_ref[...] = lax.dot_general(
        x_ref[...], w_ref[...], (((1,), (1,)), ((), ())),
        preferred_element_type=jnp.float32)


def _project(x, W):
    return pl.pallas_call(
        _matmul_body,
        grid=(10,),
        in_specs=[
            pl.BlockSpec((N // 10, D), lambda i: (i, 0)),
            pl.BlockSpec((D, D), lambda i: (0, 0)),
        ],
        out_specs=pl.BlockSpec((N // 10, D), lambda i: (i, 0)),
        out_shape=jax.ShapeDtypeStruct((N, D), jnp.float32),
    )(x, W)


_mesh = plsc.VectorSubcoreMesh(
    core_axis_name="c", subcore_axis_name="s", num_cores=NC, num_subcores=NS)


@functools.partial(
    pl.kernel,
    out_type=jax.ShapeDtypeStruct((NC, N, D), jnp.float32),
    mesh=_mesh,
    scratch_types=[
        pltpu.VMEM((IB, CH), jnp.int32),
        pltpu.VMEM((IB, CH), jnp.int32),
        pltpu.VMEM((1, TE), jnp.int32),
        pltpu.VMEM((1, TE), jnp.int32),
        pltpu.VMEM((CH, D), jnp.float32),
        pltpu.VMEM((CH, D), jnp.float32),
        pltpu.VMEM((CH, D), jnp.float32),
        pltpu.VMEM((CH, D), jnp.float32),
        pltpu.VMEM_SHARED((N, D), jnp.float32),
        pltpu.SemaphoreType.DMA,
        pltpu.SemaphoreType.DMA,
        pltpu.SemaphoreType.DMA,
        pltpu.SemaphoreType.DMA,
        pltpu.SemaphoreType.DMA,
        pltpu.SemaphoreType.DMA,
    ],
)
def _message_pass(xw_hbm, srcm_hbm, dstm_hbm, srct_hbm, dstt_hbm, ea_hbm,
                  out_hbm, src_v, dst_v, srct_v, dstt_v, xj0, xj1, em0,
                  em1, acc, g0, g1, e0, e1, s0, s1):
    c = lax.axis_index("c")
    s = lax.axis_index("s")
    wid = s * NC + c
    ebase = wid * EPW

    xj = (xj0, xj1)
    em = (em0, em1)
    gsem = (g0, g1)
    esem = (e0, e1)
    ssem = (s0, s1)

    zero16 = jnp.zeros((16,), jnp.float32)

    def zero_body(r, zcarry):
        for k in range(D // 16):
            em0[r, pl.ds(k * 16, 16)] = zero16
        return zcarry

    lax.fori_loop(0, CH, zero_body, 0)
    for i in range(RPT // CH):
        pltpu.sync_copy(em0, acc.at[pl.ds(s * RPT + i * CH, CH)])
    rem = RPT - (RPT // CH) * CH
    pltpu.sync_copy(em0.at[pl.ds(0, rem)],
                    acc.at[pl.ds(s * RPT + RPT - rem, rem)])

    @pl.when(s == 0)
    def _():
        pltpu.sync_copy(em0.at[pl.ds(0, TAIL)],
                        acc.at[pl.ds(NS * RPT, TAIL)])

    plsc.subcore_barrier()

    def _wait_gather(a):
        pltpu.make_async_copy(
            xw_hbm.at[pl.ds(0, CH)], xj[a], gsem[a]).wait()

    def _wait_ea(a):
        pltpu.make_async_copy(
            ea_hbm.at[pl.ds(0, CH)], em[a], esem[a]).wait()

    def _wait_scatter(a):
        pltpu.make_async_copy(
            em[a], acc.at[pl.ds(0, CH)], ssem[a]).wait()

    def _issue_ea(j, a):
        pltpu.async_copy(ea_hbm.at[pl.ds(ebase + j * CH, CH)],
                         em[a], esem[a])

    def blk_body(bi, bcarry):
        j0 = bi * IB

        @pl.when(bi > 0)
        def _():
            _wait_scatter(1)

        pltpu.sync_copy(srcm_hbm.at[wid, bi], src_v)
        pltpu.sync_copy(dstm_hbm.at[wid, bi], dst_v)

        pltpu.async_copy(xw_hbm.at[src_v.at[0]], xj0, g0)

        @pl.when(bi == 0)
        def _():
            _issue_ea(0, 0)

        def pair_body(pair, carry):
            for a in (0, 1):
                jj = 2 * pair + a
                j = j0 + jj
                o = 1 - a

                if a == 0:
                    pltpu.async_copy(xw_hbm.at[src_v.at[jj + 1]],
                                     xj[o], gsem[o])
                else:
                    @pl.when(pair < NP - 1)
                    def _():
                        pltpu.async_copy(xw_hbm.at[src_v.at[jj + 1]],
                                         xj[o], gsem[o])

                _wait_gather(a)
                _wait_ea(a)

                xja, ema = xj[a], em[a]

                def row_body(r, rcarry):
                    for k in range(D // 16):
                        sl = pl.ds(k * 16, 16)
                        ema[r, sl] = jnp.maximum(
                            xja[r, sl] + ema[r, sl], 0.0)
                    return rcarry

                lax.fori_loop(0, CH, row_body, 0)

                if a == 0:
                    @pl.when(pair > 0)
                    def _():
                        _wait_scatter(o)
                else:
                    _wait_scatter(o)

                @pl.when(j + 1 < NCH)
                def _():
                    _issue_ea(j + 1, o)

                pltpu.async_copy(em[a], acc.at[dst_v.at[jj]],
                                 ssem[a], add=True)
            return carry

        lax.fori_loop(0, NP, pair_body, 0)
        return bcarry

    lax.fori_loop(0, NIB, blk_body, 0)
    _wait_scatter(1)

    pltpu.sync_copy(srct_hbm.at[wid], srct_v)
    pltpu.sync_copy(dstt_hbm.at[wid], dstt_v)
    tg = pltpu.async_copy(xw_hbm.at[srct_v.at[0]],
                          xj0.at[pl.ds(0, TE)], g0)
    pltpu.sync_copy(ea_hbm.at[pl.ds(ebase + NCH * CH, TE)],
                    em0.at[pl.ds(0, TE)])
    tg.wait()

    def tail_body(r, rcarry):
        for k in range(D // 16):
            sl = pl.ds(k * 16, 16)
            em0[r, sl] = jnp.maximum(xj0[r, sl] + em0[r, sl], 0.0)
        return rcarry

    lax.fori_loop(0, TE, tail_body, 0)
    pltpu.sync_copy(em0.at[pl.ds(0, TE)], acc.at[dstt_v.at[0]], add=True)
    plsc.subcore_barrier()

    pltpu.sync_copy(acc.at[pl.ds(s * RPT, RPT)],
                    out_hbm.at[c, pl.ds(s * RPT, RPT)])

    @pl.when(s == 0)
    def _():
        pltpu.sync_copy(acc.at[pl.ds(NS * RPT, TAIL)],
                        out_hbm.at[c, pl.ds(NS * RPT, TAIL)])


def _combine_body(p_ref, b_ref, o_ref):
    o_ref[...] = p_ref[0] + p_ref[1] + b_ref[...]


def _combine(partials, b2d):
    return pl.pallas_call(
        _combine_body,
        grid=(10,),
        in_specs=[
            pl.BlockSpec((NC, N // 10, D), lambda i: (0, i, 0)),
            pl.BlockSpec((1, D), lambda i: (0, 0)),
        ],
        out_specs=pl.BlockSpec((N // 10, D), lambda i: (i, 0)),
        out_shape=jax.ShapeDtypeStruct((N, D), jnp.float32),
    )(partials, b2d)


def kernel(x, edge_index, edge_attr, W, b):
    src = edge_index[0].reshape(NW, EPW)
    dst = edge_index[1].reshape(NW, EPW)
    srcm = src[:, :NCH * CH].reshape(NW, NIB, IB, CH)
    dstm = dst[:, :NCH * CH].reshape(NW, NIB, IB, CH)
    srct = src[:, NCH * CH:].reshape(NW, 1, TE)
    dstt = dst[:, NCH * CH:].reshape(NW, 1, TE)
    xw = _project(x, W)
    partials = _message_pass(xw, srcm, dstm, srct, dstt, edge_attr)
    return _combine(partials, b.reshape(1, D))
